# baseline (speedup 1.0000x reference)
import functools

import jax
import jax.numpy as jnp
from jax import lax
from jax.experimental import pallas as pl
from jax.experimental.pallas import tpu as pltpu
from jax.experimental.pallas import tpu_sc as plsc

N = 10000
D = 128
E2 = 640000
NC = 2
NS = 16
NW = NC * NS
M = E2 // NW
CW = 50
NCHUNK = M // CW
NB = 5
IG = 8
PG = 8
NPAD = 10240
RPT = NPAD // NS
RC = 40

_mesh = plsc.VectorSubcoreMesh(core_axis_name="c", subcore_axis_name="s")
_f32 = jnp.float32


def _zero_rows(buf, nrows):
    def row(r, _):
        for j in range(D // 16):
            buf[r, pl.ds(j * 16, 16)] = jnp.zeros((16,), _f32)
        return ()
    lax.fori_loop(0, nrows, row, ())


@functools.partial(
    pl.kernel,
    out_type=jax.ShapeDtypeStruct((2 * NPAD,), _f32),
    mesh=_mesh,
    scratch_types=[
        pltpu.VMEM((IG, CW), jnp.int32),
        pltpu.VMEM((128,), _f32),
        pltpu.VMEM((640,), _f32),
        pltpu.VMEM_SHARED((NPAD,), _f32),
    ],
)
def _hist_kernel(dst_hbm, hist_hbm, idxb, ones, zb, hist):
    c = lax.axis_index("c")
    s = lax.axis_index("s")

    for i in range(40):
        zb[pl.ds(i * 16, 16)] = jnp.zeros((16,), _f32)
    for i in range(8):
        ones[pl.ds(i * 16, 16)] = jnp.full((16,), 1.0, _f32)
    pltpu.sync_copy(zb, hist.at[pl.ds(s * 640, 640)])
    plsc.subcore_barrier()

    base_row = (c * NS + s) * NCHUNK

    def hbody(j8, _):
        pltpu.sync_copy(dst_hbm.at[pl.ds(base_row + j8 * IG, IG)], idxb)
        for jj in range(IG):
            pltpu.sync_copy(ones.at[pl.ds(0, CW)], hist.at[idxb.at[jj]],
                            add=True)
        return ()
    lax.fori_loop(0, NCHUNK // IG, hbody, ())
    plsc.subcore_barrier()

    pltpu.sync_copy(hist.at[pl.ds(s * 640, 640)],
                    hist_hbm.at[pl.ds(c * NPAD + s * 640, 640)])


def _dg_body(h0, h1, emb, dinv, g0):
    h = h0[...] + h1[...]
    d = jnp.where(h > 0.5, lax.rsqrt(jnp.maximum(h, 1.0)), 0.0)
    dinv[...] = d
    g0[...] = d * emb[...]


def _dinv_g0(hist0, hist1, emb):
    return pl.pallas_call(
        _dg_body,
        grid=(N // _BU,),
        in_specs=[_row_spec(1), _row_spec(1), _row_spec(D)],
        out_specs=[_row_spec(1), _row_spec(D)],
        out_shape=[
            jax.ShapeDtypeStruct((N, 1), _f32),
            jax.ShapeDtypeStruct((N, D), _f32),
        ],
    )(hist0, hist1, emb)


@functools.partial(
    pl.kernel,
    out_type=(
        jax.ShapeDtypeStruct((NPAD, D), _f32),
        jax.ShapeDtypeStruct((NPAD, D), _f32),
    ),
    mesh=_mesh,
    scratch_types=[
        pltpu.VMEM((2, PG, CW), jnp.int32),
        pltpu.VMEM((2, PG, CW), jnp.int32),
        pltpu.VMEM((CW, D), _f32),
        pltpu.VMEM((CW, D), _f32),
        pltpu.VMEM((CW, D), _f32),
        pltpu.VMEM((CW, D), _f32),
        pltpu.VMEM((CW, D), _f32),
        pltpu.SemaphoreType.DMA,
        pltpu.SemaphoreType.DMA,
        pltpu.SemaphoreType.DMA,
        pltpu.SemaphoreType.DMA,
        pltpu.SemaphoreType.DMA,
        pltpu.SemaphoreType.DMA,
        pltpu.SemaphoreType.DMA,
        pltpu.SemaphoreType.DMA,
        pltpu.SemaphoreType.DMA,
        pltpu.SemaphoreType.DMA,
        pltpu.SemaphoreType.DMA,
        pltpu.SemaphoreType.DMA,
        pltpu.VMEM_SHARED((NPAD, D), _f32),
    ],
)
def _prop(g_hbm, dst_hbm, src_hbm, p0_hbm, p1_hbm,
          dstb, srcb, b0, b1, b2, b3, b4, g0s, g1s, g2s, g3s, g4s,
          s0s, s1s, s2s, s3s, s4s, i0s, i1s, acc):
    c = lax.axis_index("c")
    s = lax.axis_index("s")
    w = c * NS + s

    _zero_rows(b0, CW)
    for k in range(RPT // RC):
        pltpu.sync_copy(b0.at[pl.ds(0, RC)],
                        acc.at[pl.ds(s * RPT + k * RC, RC)])
    plsc.subcore_barrier()

    base_row = w * NCHUNK
    bufs = (b0, b1, b2, b3, b4)
    gsems = (g0s, g1s, g2s, g3s, g4s)
    ssems = (s0s, s1s, s2s, s3s, s4s)
    isems = (i0s, i1s)
    NG = NCHUNK // PG
    pend_g = [None] * NB
    pend_s = [None] * NB
    pend_i = [None, None]

    def load_idx(g, sync):
        slot = g % 2
        r0 = base_row + g * PG
        if sync:
            pltpu.sync_copy(dst_hbm.at[pl.ds(r0, PG)], dstb.at[slot])
            pltpu.sync_copy(src_hbm.at[pl.ds(r0, PG)], srcb.at[slot])
        else:
            pend_i[slot] = (
                pltpu.async_copy(dst_hbm.at[pl.ds(r0, PG)], dstb.at[slot],
                                 isems[slot]),
                pltpu.async_copy(src_hbm.at[pl.ds(r0, PG)], srcb.at[slot],
                                 isems[slot]),
            )

    def issue_gather(j):
        g, jj = divmod(j, PG)
        p = j % NB
        if pend_s[p] is not None:
            pend_s[p].wait()
            pend_s[p] = None
        if jj == 0 and pend_i[g % 2] is not None:
            for d in pend_i[g % 2]:
                d.wait()
            pend_i[g % 2] = None
        pend_g[p] = pltpu.async_copy(
            g_hbm.at[srcb.at[g % 2, jj]], bufs[p], gsems[p])

    load_idx(0, sync=True)
    if NG > 1:
        load_idx(1, sync=False)
    for j in range(NB - 1):
        issue_gather(j)
    for k in range(NCHUNK):
        g, jj = divmod(k, PG)
        p = k % NB
        if k + NB - 1 < NCHUNK:
            issue_gather(k + NB - 1)
        if jj == 2 and 2 <= g + 1 < NG:
            load_idx(g + 1, sync=False)
        pend_g[p].wait()
        pend_g[p] = None
        pend_s[p] = pltpu.async_copy(
            bufs[p], acc.at[dstb.at[g % 2, jj]], ssems[p], add=True)
    for p in range(NB):
        if pend_s[p] is not None:
            pend_s[p].wait()
            pend_s[p] = None
    plsc.subcore_barrier()

    @pl.when(c == 0)
    def _():
        for k in range(RPT // RC):
            r = s * RPT + k * RC
            pltpu.sync_copy(acc.at[pl.ds(r, RC)], p0_hbm.at[pl.ds(r, RC)])

    @pl.when(c == 1)
    def _():
        for k in range(RPT // RC):
            r = s * RPT + k * RC
            pltpu.sync_copy(acc.at[pl.ds(r, RC)], p1_hbm.at[pl.ds(r, RC)])


def _scale_body(p0, p1, dinv, g):
    d = dinv[...]
    g[...] = (d * d) * (p0[...] + p1[...])


def _scale(p0, p1, dinv2):
    return pl.pallas_call(
        _scale_body,
        grid=(N // _BU,),
        in_specs=[_row_spec(D), _row_spec(D), _row_spec(1)],
        out_specs=_row_spec(D),
        out_shape=jax.ShapeDtypeStruct((N, D), _f32),
    )(p0, p1, dinv2)


def _heads_body(u, p01, p11, p02, p12, p03, p13, dinv,
                cw1, cb1, cw2, cb2, aw1, ab1, aw2, ab2, sw1, sb1, sw2, sb2,
                churn, cat, sku, uf):
    psum = (p01[...] + p11[...] + p02[...] + p12[...] + p03[...] + p13[...])
    x = (u[...] + dinv[...] * psum) * 0.25
    uf[...] = x

    def head(w1, b1, w2, b2):
        h = jnp.maximum(
            jnp.dot(x, w1[...], preferred_element_type=jnp.float32) + b1[...],
            0.0)
        return jax.nn.sigmoid(
            jnp.dot(h, w2[...], preferred_element_type=jnp.float32) + b2[...])

    churn[...] = head(cw1, cb1, cw2, cb2)
    cat[...] = head(aw1, ab1, aw2, ab2)
    sku[...] = head(sw1, sb1, sw2, sb2)


_BU = 1000


def _row_spec(cols):
    return pl.BlockSpec((_BU, cols), lambda i: (i, 0))


def _full_spec(r, cols):
    return pl.BlockSpec((r, cols), lambda i: (0, 0))


def _heads(u, p01, p11, p02, p12, p03, p13, dinv2,
           cw1, cb1, cw2, cb2, aw1, ab1, aw2, ab2, sw1, sb1, sw2, sb2):
    nu = u.shape[0]
    return pl.pallas_call(
        _heads_body,
        grid=(nu // _BU,),
        in_specs=[
            _row_spec(D),
            _row_spec(D), _row_spec(D), _row_spec(D),
            _row_spec(D), _row_spec(D), _row_spec(D),
            _row_spec(1),
            _full_spec(D, 128), _full_spec(1, 128),
            _full_spec(128, 1), _full_spec(1, 1),
            _full_spec(D, 128), _full_spec(1, 128),
            _full_spec(128, 100), _full_spec(1, 100),
            _full_spec(D, 128), _full_spec(1, 128),
            _full_spec(128, 1000), _full_spec(1, 1000),
        ],
        out_specs=[
            _row_spec(1), _row_spec(100), _row_spec(1000), _row_spec(D),
        ],
        out_shape=[
            jax.ShapeDtypeStruct((nu, 1), _f32),
            jax.ShapeDtypeStruct((nu, 100), _f32),
            jax.ShapeDtypeStruct((nu, 1000), _f32),
            jax.ShapeDtypeStruct((nu, D), _f32),
        ],
    )(u, p01, p11, p02, p12, p03, p13, dinv2,
      cw1, cb1, cw2, cb2, aw1, ab1, aw2, ab2, sw1, sb1, sw2, sb2)


def kernel(user_emb_w, item_emb_w, churn_w1, churn_b1, churn_w2, churn_b2,
           cat_w1, cat_b1, cat_w2, cat_b2, sku_w1, sku_b1, sku_w2, sku_b2,
           edge_index):
    ei = edge_index.astype(jnp.int32)
    dst = jnp.concatenate([ei[0], ei[1]]).reshape(NW * NCHUNK, CW)
    src = jnp.concatenate([ei[1], ei[0]]).reshape(NW * NCHUNK, CW)
    emb = jnp.concatenate([user_emb_w, item_emb_w], axis=0)

    hist = _hist_kernel(dst)
    dinv2f, g0 = _dinv_g0(hist[:N].reshape(N, 1),
                          hist[NPAD:NPAD + N].reshape(N, 1), emb)
    p01, p11 = _prop(g0, dst, src)
    g1 = _scale(p01, p11, dinv2f)
    p02, p12 = _prop(g1, dst, src)
    g2 = _scale(p02, p12, dinv2f)
    p03, p13 = _prop(g2, dst, src)

    nu = user_emb_w.shape[0]
    dinv2 = dinv2f[:nu]
    churn, cat, sku, uf = _heads(
        user_emb_w, p01, p11, p02, p12, p03, p13, dinv2,
        churn_w1, churn_b1.reshape(1, 128), churn_w2, churn_b2.reshape(1, 1),
        cat_w1, cat_b1.reshape(1, 128), cat_w2, cat_b2.reshape(1, 100),
        sku_w1, sku_b1.reshape(1, 128), sku_w2, sku_b2.reshape(1, 1000))
    return churn, cat, sku, uf
